# baseline (speedup 1.0000x reference)
import functools

import jax
import jax.numpy as jnp
from jax import lax
from jax.experimental import pallas as pl
from jax.experimental.pallas import tpu as pltpu
from jax.experimental.pallas import tpu_sc as plsc

B = 16384
F = 16
NC = 2
NS = 16
NW = NC * NS
BPW = B // NW
CHUNK = 128
NCHUNK = BPW // CHUNK

_MESH = plsc.VectorSubcoreMesh(core_axis_name="c", subcore_axis_name="s")


@functools.partial(
    pl.kernel,
    out_type=[
        jax.ShapeDtypeStruct((B, F), jnp.float32),
        jax.ShapeDtypeStruct((B, F), jnp.float32),
    ],
    mesh=_MESH,
    compiler_params=pltpu.CompilerParams(use_tc_tiling_on_sc=False),
    scratch_types=[
        pltpu.VMEM((BPW,), jnp.int32),
        pltpu.VMEM((BPW,), jnp.int32),
        pltpu.VMEM((BPW, F), jnp.float32),
        pltpu.VMEM((BPW, F), jnp.float32),
        pltpu.SemaphoreType.DMA,
    ],
)
def _sc_gather(user_h, movie_h, ut_h, mt_h, uo_h, mo_h,
               uidx_v, midx_v, urows_v, mrows_v, sem):
    wid = lax.axis_index("s") * NC + lax.axis_index("c")
    base = wid * BPW
    pltpu.sync_copy(user_h.at[pl.ds(base, BPW)], uidx_v)
    pltpu.sync_copy(movie_h.at[pl.ds(base, BPW)], midx_v)
    copies = []
    for j in range(NCHUNK):
        sl = pl.ds(j * CHUNK, CHUNK)
        copies.append(pltpu.async_copy(ut_h.at[uidx_v.at[sl]], urows_v.at[sl], sem))
        copies.append(pltpu.async_copy(mt_h.at[midx_v.at[sl]], mrows_v.at[sl], sem))
    for c in copies:
        c.wait()
    pltpu.sync_copy(urows_v, uo_h.at[pl.ds(base, BPW)])
    pltpu.sync_copy(mrows_v, mo_h.at[pl.ds(base, BPW)])


def _mlp_body(u_ref, m_ref, w1u_ref, w1m_ref, b1_ref, w2_ref, b2_ref, o_ref):
    h = jnp.dot(u_ref[...], w1u_ref[...], preferred_element_type=jnp.float32)
    h = h + jnp.dot(m_ref[...], w1m_ref[...], preferred_element_type=jnp.float32)
    h = jnp.maximum(h + b1_ref[...], 0.0)
    o = jnp.dot(h, w2_ref[...], preferred_element_type=jnp.float32) + b2_ref[...]
    o_ref[...] = 5.5 / (1.0 + jnp.exp(-o))


def _mlp(u_emb, m_emb, w1u, w1m, b1, w2, b2):
    return pl.pallas_call(
        _mlp_body,
        out_shape=jax.ShapeDtypeStruct((B, 1), jnp.float32),
    )(u_emb, m_emb, w1u, w1m, b1, w2, b2)


def kernel(user, movie, u_table, m_table, W1, b1, W2, b2):
    user = user.astype(jnp.int32)
    movie = movie.astype(jnp.int32)
    u_emb, m_emb = _sc_gather(user, movie, u_table, m_table)
    return _mlp(u_emb, m_emb, W1[:F], W1[F:], b1.reshape(1, -1), W2,
                b2.reshape(1, 1))
